# baseline (speedup 1.0000x reference)
import jax
import jax.numpy as jnp
from jax import lax
from jax.experimental import pallas as pl
from jax.experimental.pallas import tpu as pltpu
from jax.experimental.pallas import tpu_sc as plsc

NUM_ROWS_TBL = 100000
BATCH_N = 16384
LR = 200
LS = 50
NC = 2
NS = 16
NW = NC * NS
ROWS_PER_W = BATCH_N // NW
GROUPS = ROWS_PER_W // 16
GPC = 2
CHUNKS = GROUPS // GPC
SGPC = 2
SCHUNKS = GROUPS // SGPC
SCH = SGPC * 16 * LS


def _sc_body(rule_2d, syn_flat, rw_hbm, sw_hbm, gb_hbm, out_hbm,
             table_v, r2d_v0, r2d_v1, rflat_v, sidx_v0, sidx_v1,
             acc_v, gb_v, sem0, sem1, semt):
    wid = lax.axis_index("s") * NC + lax.axis_index("c")
    base = wid * ROWS_PER_W

    lane = jnp.arange(16, dtype=jnp.int32)
    zero16 = jnp.zeros((16,), jnp.float32)
    sems = (sem0, sem1)
    r2ds = (r2d_v0, r2d_v1)
    sbufs = (sidx_v0, sidx_v1)
    lane_r = lane * LR
    lane_s = lane * LS

    def rule_dma(c, slot):
        row0 = pl.multiple_of(base + c * (GPC * 16), 8)
        return pltpu.async_copy(rule_2d.at[pl.ds(row0, GPC * 16), :],
                                r2ds[slot], sems[slot])

    def syn_dma(c, slot):
        off = pl.multiple_of(base * LS + c * SCH, 8)
        return pltpu.async_copy(syn_flat.at[pl.ds(off, SCH)],
                                sbufs[slot], sems[slot])

    pending = rule_dma(0, 0)
    tdma = pltpu.async_copy(rw_hbm.at[pl.ds(0, NUM_ROWS_TBL)], table_v, semt)
    pltpu.sync_copy(gb_hbm, gb_v)
    gamma = gb_v[pl.ds(0, 16)]
    bias = gb_v[pl.ds(16, 16)]
    tdma.wait()

    j0s = list(range(0, LR - 15, 16))
    if LR % 16:
        j0s.append(LR - 16)

    def repack(src2d):
        @plsc.parallel_loop(0, GPC * 16, unroll=2)
        def _row(r):
            rfull = jnp.full((16,), r, jnp.int32)
            for j0 in j0s:
                v = plsc.load_gather(src2d, [rfull, lane + j0])
                rflat_v[pl.ds(r * LR + j0, 16)] = v

    for c in range(CHUNKS):
        pending.wait()
        if c + 1 < CHUNKS:
            pending = rule_dma(c + 1, (c + 1) % 2)
        repack(r2ds[c % 2])
        for k in range(GPC):
            g = c * GPC + k
            rbuf = rflat_v.at[pl.ds(k * 16 * LR, 16 * LR)]

            @plsc.parallel_loop(0, LR, unroll=8, carry=zero16)
            def _racc(j, acc, rbuf=rbuf):
                col = plsc.load_gather(rbuf, [lane_r + j])
                w = plsc.load_gather(table_v, [col])
                return acc + w * w

            acc_v[pl.ds(g * 16, 16)] = _racc

    pending = syn_dma(0, 0)
    pltpu.sync_copy(sw_hbm.at[pl.ds(0, NUM_ROWS_TBL)], table_v)

    for c in range(SCHUNKS):
        pending.wait()
        if c + 1 < SCHUNKS:
            pending = syn_dma(c + 1, (c + 1) % 2)
        for k in range(SGPC):
            g = c * SGPC + k
            sbuf = sbufs[c % 2].at[pl.ds(k * 16 * LS, 16 * LS)]

            @plsc.parallel_loop(0, LS, unroll=5,
                                carry=acc_v[pl.ds(g * 16, 16)])
            def _sacc(j, acc, sbuf=sbuf):
                col = plsc.load_gather(sbuf, [lane_s + j])
                w = plsc.load_gather(table_v, [col])
                return acc + w * w

            score = gamma * _sacc + bias
            score = jnp.minimum(jnp.maximum(score, 0.0), 30.0)
            acc_v[pl.ds(g * 16, 16)] = 1.0 - jnp.exp(-score)

    pltpu.sync_copy(acc_v, out_hbm.at[pl.ds(base, ROWS_PER_W)])


@jax.jit
def _surprisal_sc(rule_2d, syn_flat, rw, sw, gb):
    mesh = plsc.VectorSubcoreMesh(core_axis_name="c", subcore_axis_name="s",
                                  num_cores=NC, num_subcores=NS)
    return pl.kernel(
        _sc_body,
        out_type=jax.ShapeDtypeStruct((BATCH_N,), jnp.float32),
        mesh=mesh,
        compiler_params=pltpu.CompilerParams(needs_layout_passes=False),
        scratch_types=[
            pltpu.VMEM((NUM_ROWS_TBL,), jnp.float32),
            pltpu.VMEM((GPC * 16, LR), jnp.int32),
            pltpu.VMEM((GPC * 16, LR), jnp.int32),
            pltpu.VMEM((GPC * 16 * LR,), jnp.int32),
            pltpu.VMEM((SCH,), jnp.int32),
            pltpu.VMEM((SCH,), jnp.int32),
            pltpu.VMEM((ROWS_PER_W,), jnp.float32),
            pltpu.VMEM((32,), jnp.float32),
            pltpu.SemaphoreType.DMA,
            pltpu.SemaphoreType.DMA,
            pltpu.SemaphoreType.DMA,
        ],
    )(rule_2d, syn_flat, rw, sw, gb)


def kernel(rule_idx, synergy_idx, rules_w, synergy_w, bias, gamma):
    syn_flat = synergy_idx.astype(jnp.int32).reshape(-1)
    gb = jnp.concatenate([jnp.broadcast_to(gamma, (16,)),
                          jnp.broadcast_to(bias, (16,))])
    return _surprisal_sc(rule_idx.astype(jnp.int32), syn_flat,
                         rules_w.reshape(-1), synergy_w.reshape(-1), gb)
